# Initial kernel scaffold; baseline (speedup 1.0000x reference)
import jax
import jax.numpy as jnp
from jax import lax
from jax.experimental import pallas as pl
from jax.experimental.pallas import tpu as pltpu
from jax.experimental.pallas import tpu_sc as plsc

NUM_COLLISIONS = 4
EMBED_DIM = 64
BATCH = 16384
BAG = 20

NC = 2
NS = 16
NW = NC * NS
LANES = 16

BAGS_PER_W = BATCH // NW
CHUNK = 32
NCHUNK = BAGS_PER_W // CHUNK
ROWS_PER_CHUNK = CHUNK * BAG
IDX_GROUPS = ROWS_PER_CHUNK // 128
DSLICES = EMBED_DIM // LANES


def _er_body(idx_ref, wr_ref, o_ref):
    r = idx_ref[...] & 3
    wr = wr_ref[...]
    acc = jnp.zeros((idx_ref.shape[0], EMBED_DIM), jnp.float32)
    for k in range(NUM_COLLISIONS):
        cnt = jnp.sum((r == k).astype(jnp.float32), axis=1, keepdims=True)
        acc = acc + cnt * wr[k:k + 1, :]
    o_ref[...] = acc * (1.0 / (BAG * BAG))


def _sc_body(inp_hbm, wq_hbm, er_hbm, out_hbm,
             idx_raw, idx_q, rows, er_v, out_v, sem):
    wid = lax.axis_index("s") * NC + lax.axis_index("c")
    bag_base = wid * BAGS_PER_W

    def chunk_body(c, carry):
        cb = bag_base + c * CHUNK
        rb = cb * BAG
        pltpu.sync_copy(inp_hbm.at[pl.ds(rb, ROWS_PER_CHUNK)], idx_raw)
        pltpu.sync_copy(er_hbm.at[pl.ds(cb, CHUNK)], er_v)

        for g in range(IDX_GROUPS):
            def q_body(j, _, g=g):
                v = idx_raw[pl.ds(g * 128 + j * LANES, LANES)]
                idx_q[g, pl.ds(j * LANES, LANES)] = jnp.right_shift(v, 2)
                return 0
            lax.fori_loop(0, 128 // LANES, q_body, 0)

        copies = [
            pltpu.async_copy(wq_hbm.at[idx_q.at[g]],
                             rows.at[pl.ds(g * 128, 128)], sem)
            for g in range(IDX_GROUPS)
        ]
        for cp in copies:
            cp.wait()

        def bag_body(b, _):
            rbase = b * BAG

            def j_body(j, accs):
                rr = rbase + j
                return tuple(accs[s] + rows[rr, pl.ds(s * LANES, LANES)]
                             for s in range(DSLICES))

            z = jnp.zeros((LANES,), jnp.float32)
            accs = lax.fori_loop(0, BAG, j_body, (z,) * DSLICES)
            for s in range(DSLICES):
                out_v[b, pl.ds(s * LANES, LANES)] = (
                    accs[s] * er_v[b, pl.ds(s * LANES, LANES)])
            return 0

        lax.fori_loop(0, CHUNK, bag_body, 0)
        pltpu.sync_copy(out_v, out_hbm.at[pl.ds(cb, CHUNK)])
        return carry

    lax.fori_loop(0, NCHUNK, chunk_body, 0)


_sc_call = pl.kernel(
    _sc_body,
    out_type=jax.ShapeDtypeStruct((BATCH, EMBED_DIM), jnp.float32),
    mesh=plsc.VectorSubcoreMesh(core_axis_name="c", subcore_axis_name="s"),
    scratch_types=[
        pltpu.VMEM((ROWS_PER_CHUNK,), jnp.int32),
        pltpu.VMEM((IDX_GROUPS, 128), jnp.int32),
        pltpu.VMEM((ROWS_PER_CHUNK, EMBED_DIM), jnp.float32),
        pltpu.VMEM((CHUNK, EMBED_DIM), jnp.float32),
        pltpu.VMEM((CHUNK, EMBED_DIM), jnp.float32),
        pltpu.SemaphoreType.DMA,
    ],
)


def kernel(input, weight_q, weight_r):
    idx = input.astype(jnp.int32)
    blk = 2048
    er = pl.pallas_call(
        _er_body,
        grid=(BATCH // blk,),
        in_specs=[
            pl.BlockSpec((blk, BAG), lambda i: (i, 0)),
            pl.BlockSpec((NUM_COLLISIONS, EMBED_DIM), lambda i: (0, 0)),
        ],
        out_specs=pl.BlockSpec((blk, EMBED_DIM), lambda i: (i, 0)),
        out_shape=jax.ShapeDtypeStruct((BATCH, EMBED_DIM), jnp.float32),
    )(idx, weight_r)
    return _sc_call(idx.reshape(BATCH * BAG), weight_q, er)


# speedup vs baseline: 5.9998x; 5.9998x over previous
import jax
import jax.numpy as jnp
from jax import lax
from jax.experimental import pallas as pl
from jax.experimental.pallas import tpu as pltpu
from jax.experimental.pallas import tpu_sc as plsc

NUM_COLLISIONS = 4
EMBED_DIM = 64
BATCH = 16384
BAG = 20

NC = 2
NS = 16
NW = NC * NS
LANES = 16

BAGS_PER_W = BATCH // NW
CHUNK = 32
NCHUNK = BAGS_PER_W // CHUNK
ROWS_PER_CHUNK = CHUNK * BAG
IDX_GROUPS = ROWS_PER_CHUNK // 128
DSLICES = EMBED_DIM // LANES


def _er_body(idx_ref, wr_ref, o_ref):
    r = idx_ref[...] & 3
    wr = wr_ref[...]
    acc = jnp.zeros((idx_ref.shape[0], EMBED_DIM), jnp.float32)
    for k in range(NUM_COLLISIONS):
        cnt = jnp.sum((r == k).astype(jnp.float32), axis=1, keepdims=True)
        acc = acc + cnt * wr[k:k + 1, :]
    o_ref[...] = acc * (1.0 / (BAG * BAG))


def _sc_body(inp_hbm, wq_hbm, er_hbm, out_hbm,
             idx_raw, idx_q, rows, er_v, out_v, sem):
    wid = lax.axis_index("s") * NC + lax.axis_index("c")
    bag_base = wid * BAGS_PER_W

    def chunk_body(c, carry):
        cb = bag_base + c * CHUNK
        rb = cb * BAG
        pltpu.sync_copy(inp_hbm.at[pl.ds(rb, ROWS_PER_CHUNK)], idx_raw)
        pltpu.sync_copy(er_hbm.at[pl.ds(cb, CHUNK)], er_v)

        for g in range(IDX_GROUPS):
            def q_body(j, _, g=g):
                v = idx_raw[pl.ds(g * 128 + j * LANES, LANES)]
                idx_q[g, pl.ds(j * LANES, LANES)] = jnp.right_shift(v, 2)
                return 0
            lax.fori_loop(0, 128 // LANES, q_body, 0)

        copies = [
            pltpu.async_copy(wq_hbm.at[idx_q.at[g]],
                             rows.at[pl.ds(g * 128, 128)], sem)
            for g in range(IDX_GROUPS)
        ]
        for cp in copies:
            cp.wait()

        def bag_body(b, _):
            rbase = b * BAG

            def j_body(j, accs):
                rr = rbase + j
                return tuple(accs[s] + rows[rr, pl.ds(s * LANES, LANES)]
                             for s in range(DSLICES))

            z = jnp.zeros((LANES,), jnp.float32)
            accs = lax.fori_loop(0, BAG, j_body, (z,) * DSLICES)
            for s in range(DSLICES):
                out_v[b, pl.ds(s * LANES, LANES)] = (
                    accs[s] * er_v[b, pl.ds(s * LANES, LANES)])
            return 0

        lax.fori_loop(0, CHUNK, bag_body, 0)
        pltpu.sync_copy(out_v, out_hbm.at[pl.ds(cb, CHUNK)])
        return carry

    lax.fori_loop(0, NCHUNK, chunk_body, 0)


_sc_call = pl.kernel(
    _sc_body,
    out_type=jax.ShapeDtypeStruct((BATCH, EMBED_DIM), jnp.float32),
    mesh=plsc.VectorSubcoreMesh(core_axis_name="c", subcore_axis_name="s"),
    scratch_types=[
        pltpu.VMEM((ROWS_PER_CHUNK,), jnp.int32),
        pltpu.VMEM((IDX_GROUPS, 128), jnp.int32),
        pltpu.VMEM((ROWS_PER_CHUNK, EMBED_DIM), jnp.float32),
        pltpu.VMEM((CHUNK, EMBED_DIM), jnp.float32),
        pltpu.VMEM((CHUNK, EMBED_DIM), jnp.float32),
        pltpu.SemaphoreType.DMA,
    ],
    compiler_params=pltpu.CompilerParams(use_tc_tiling_on_sc=False),
)


def kernel(input, weight_q, weight_r):
    idx = input.astype(jnp.int32)
    blk = 2048
    er = pl.pallas_call(
        _er_body,
        grid=(BATCH // blk,),
        in_specs=[
            pl.BlockSpec((blk, BAG), lambda i: (i, 0)),
            pl.BlockSpec((NUM_COLLISIONS, EMBED_DIM), lambda i: (0, 0)),
        ],
        out_specs=pl.BlockSpec((blk, EMBED_DIM), lambda i: (i, 0)),
        out_shape=jax.ShapeDtypeStruct((BATCH, EMBED_DIM), jnp.float32),
    )(idx, weight_r)
    return _sc_call(idx.reshape(BATCH * BAG), weight_q, er)


# speedup vs baseline: 7.5170x; 1.2529x over previous
import jax
import jax.numpy as jnp
from jax import lax
from jax.experimental import pallas as pl
from jax.experimental.pallas import tpu as pltpu
from jax.experimental.pallas import tpu_sc as plsc

NUM_COLLISIONS = 4
EMBED_DIM = 64
BATCH = 16384
BAG = 20

NC = 2
NS = 16
NW = NC * NS
LANES = 16

BAGS_PER_W = BATCH // NW
CHUNK = 32
NCHUNK = BAGS_PER_W // CHUNK
ROWS_PER_CHUNK = CHUNK * BAG
IDX_GROUPS = ROWS_PER_CHUNK // 128
IDX_ROWS_W = BAGS_PER_W * BAG // 128
DSLICES = EMBED_DIM // LANES


def _er_body(idx_ref, wr_ref, o_ref):
    r = idx_ref[...] & 3
    wr = wr_ref[...]
    acc = jnp.zeros((idx_ref.shape[0], EMBED_DIM), jnp.float32)
    for k in range(NUM_COLLISIONS):
        cnt = jnp.sum((r == k).astype(jnp.float32), axis=1, keepdims=True)
        acc = acc + cnt * wr[k:k + 1, :]
    o_ref[...] = acc * (1.0 / (BAG * BAG))


def _sc_body(idx_hbm, wq_hbm, er_hbm, out_hbm,
             qidx, rows_a, rows_b, er_a, er_b, out_a, out_b,
             sem_a, sem_b, sem_oa, sem_ob):
    wid = lax.axis_index("s") * NC + lax.axis_index("c")
    bag_base = wid * BAGS_PER_W

    pltpu.sync_copy(idx_hbm.at[pl.ds(wid * IDX_ROWS_W, IDX_ROWS_W)], qidx)

    def shift_body(r, _):
        for cc in range(128 // LANES):
            v = qidx[r, pl.ds(cc * LANES, LANES)]
            qidx[r, pl.ds(cc * LANES, LANES)] = jnp.right_shift(v, 2)
        return 0
    lax.fori_loop(0, IDX_ROWS_W, shift_body, 0)

    def fire(c, rows_v, er_v, sem):
        for k in range(IDX_GROUPS):
            pltpu.async_copy(wq_hbm.at[qidx.at[c * IDX_GROUPS + k]],
                             rows_v.at[pl.ds(k * 128, 128)], sem)
        pltpu.async_copy(er_hbm.at[pl.ds(bag_base + c * CHUNK, CHUNK)],
                         er_v, sem)

    def wait_set(rows_v, er_v, sem):
        pltpu.make_async_copy(wq_hbm.at[pl.ds(0, ROWS_PER_CHUNK)],
                              rows_v, sem).wait()
        pltpu.make_async_copy(er_hbm.at[pl.ds(0, CHUNK)], er_v, sem).wait()

    def accum(c, rows_v, er_v, out_v, sem_o, guard):
        @pl.when(guard)
        def _():
            pltpu.make_async_copy(out_v, out_hbm.at[pl.ds(0, CHUNK)],
                                  sem_o).wait()

        def bag_body(b, _):
            rbase = b * BAG
            accs = [jnp.zeros((LANES,), jnp.float32) for _ in range(DSLICES)]
            for j in range(BAG):
                for s in range(DSLICES):
                    accs[s] = accs[s] + rows_v[rbase + j,
                                               pl.ds(s * LANES, LANES)]
            for s in range(DSLICES):
                out_v[b, pl.ds(s * LANES, LANES)] = (
                    accs[s] * er_v[b, pl.ds(s * LANES, LANES)])
            return 0

        lax.fori_loop(0, CHUNK, bag_body, 0)
        pltpu.async_copy(out_v, out_hbm.at[pl.ds(bag_base + c * CHUNK, CHUNK)],
                         sem_o)

    fire(0, rows_a, er_a, sem_a)

    def pair_body(p, _):
        c0 = 2 * p
        fire(c0 + 1, rows_b, er_b, sem_b)
        wait_set(rows_a, er_a, sem_a)
        accum(c0, rows_a, er_a, out_a, sem_oa, p > 0)

        @pl.when(p < NCHUNK // 2 - 1)
        def _():
            fire(c0 + 2, rows_a, er_a, sem_a)

        wait_set(rows_b, er_b, sem_b)
        accum(c0 + 1, rows_b, er_b, out_b, sem_ob, p > 0)
        return 0

    lax.fori_loop(0, NCHUNK // 2, pair_body, 0)
    pltpu.make_async_copy(out_a, out_hbm.at[pl.ds(0, CHUNK)], sem_oa).wait()
    pltpu.make_async_copy(out_b, out_hbm.at[pl.ds(0, CHUNK)], sem_ob).wait()


_sc_call = pl.kernel(
    _sc_body,
    out_type=jax.ShapeDtypeStruct((BATCH, EMBED_DIM), jnp.float32),
    mesh=plsc.VectorSubcoreMesh(core_axis_name="c", subcore_axis_name="s"),
    scratch_types=[
        pltpu.VMEM((NW * IDX_ROWS_W // NW, 128), jnp.int32),
        pltpu.VMEM((ROWS_PER_CHUNK, EMBED_DIM), jnp.float32),
        pltpu.VMEM((ROWS_PER_CHUNK, EMBED_DIM), jnp.float32),
        pltpu.VMEM((CHUNK, EMBED_DIM), jnp.float32),
        pltpu.VMEM((CHUNK, EMBED_DIM), jnp.float32),
        pltpu.VMEM((CHUNK, EMBED_DIM), jnp.float32),
        pltpu.VMEM((CHUNK, EMBED_DIM), jnp.float32),
        pltpu.SemaphoreType.DMA,
        pltpu.SemaphoreType.DMA,
        pltpu.SemaphoreType.DMA,
        pltpu.SemaphoreType.DMA,
    ],
    compiler_params=pltpu.CompilerParams(use_tc_tiling_on_sc=False),
)


def kernel(input, weight_q, weight_r):
    idx = input.astype(jnp.int32)
    blk = 2048
    er = pl.pallas_call(
        _er_body,
        grid=(BATCH // blk,),
        in_specs=[
            pl.BlockSpec((blk, BAG), lambda i: (i, 0)),
            pl.BlockSpec((NUM_COLLISIONS, EMBED_DIM), lambda i: (0, 0)),
        ],
        out_specs=pl.BlockSpec((blk, EMBED_DIM), lambda i: (i, 0)),
        out_shape=jax.ShapeDtypeStruct((BATCH, EMBED_DIM), jnp.float32),
    )(idx, weight_r)
    idx2d = idx.reshape(BATCH * BAG // 128, 128)
    return _sc_call(idx2d, weight_q, er)
